# baseline (speedup 1.0000x reference)
import jax
import jax.numpy as jnp
from jax import lax
from jax.experimental import pallas as pl
from jax.experimental.pallas import tpu as pltpu
from jax.experimental.pallas import tpu_sc as plsc


def _routing_sc(logits_hbm, out_hbm, lt_vmem, c_vmem):
    nc = 2
    wid = lax.axis_index("s") * nc + lax.axis_index("c")
    num_experts, _, tokens = logits_hbm.shape
    tpb = 16
    base = wid * tpb

    @pl.when(wid < tokens // tpb)
    def _work():
        for e in range(num_experts):
            pltpu.sync_copy(
                logits_hbm.at[e, 0, pl.ds(base, tpb)], lt_vmem.at[e]
            )
        fneg = jnp.full((16,), jnp.finfo(jnp.float32).min, jnp.float32)
        ineg = jnp.full((16,), -1, jnp.int32)
        one = jnp.full((16,), 1, jnp.int32)
        zero = jnp.full((16,), 0, jnp.int32)
        m1, i1, m2, i2 = fneg, ineg, fneg, ineg
        for e in range(num_experts):
            le = lt_vmem[e]
            ev = jnp.full((16,), e, jnp.int32)
            b1 = jnp.where(le > m1, one, zero)
            b2 = jnp.where(le > m2, one, zero) * (one - b1)
            m2 = jnp.where(b1 > zero, m1, jnp.where(b2 > zero, le, m2))
            i2 = jnp.where(b1 > zero, i1, jnp.where(b2 > zero, ev, i2))
            m1 = jnp.where(b1 > zero, le, m1)
            i1 = jnp.where(b1 > zero, ev, i1)
        r = jnp.exp(m2 - m1)
        w1 = 1.0 / (1.0 + r)
        w2 = r / (1.0 + r)
        for e in range(num_experts):
            ev = jnp.full((16,), e, jnp.int32)
            c_vmem[e] = jnp.where(i1 == ev, w1, 0.0) + jnp.where(i2 == ev, w2, 0.0)
        for e in range(num_experts):
            pltpu.sync_copy(
                c_vmem.at[e], out_hbm.at[e, 0, pl.ds(base, tpb)]
            )


def _routing_coeff(router_logits):
    tokens, num_experts = router_logits.shape
    logits_t = router_logits.T.reshape(num_experts, 1, tokens)
    run = pl.kernel(
        _routing_sc,
        out_type=jax.ShapeDtypeStruct((num_experts, 1, tokens), jnp.float32),
        mesh=plsc.VectorSubcoreMesh(core_axis_name="c", subcore_axis_name="s"),
        scratch_types=[
            pltpu.VMEM((num_experts, 16), jnp.float32),
            pltpu.VMEM((num_experts, 16), jnp.float32),
        ],
    )
    return run(logits_t)


def _moe_kernel(x_ref, cin_ref, w13_ref, w2_ref, out_ref):
    e = pl.program_id(0)

    @pl.when(e == 0)
    def _init():
        out_ref[...] = jnp.zeros_like(out_ref)

    x = x_ref[...]
    w13 = w13_ref[0]
    w2m = w2_ref[0]
    inter = w2m.shape[1]
    gate_up = jax.lax.dot_general(
        x, w13, (((1,), (1,)), ((), ())), preferred_element_type=jnp.float32
    )
    gate = gate_up[:, :inter]
    up = gate_up[:, inter:]
    h = gate * jax.nn.sigmoid(gate) * up
    y = jax.lax.dot_general(
        h, w2m, (((1,), (1,)), ((), ())), preferred_element_type=jnp.float32
    )
    coeff = cin_ref[0].reshape(y.shape[0], 1)
    out_ref[...] += coeff * y


def kernel(hidden_states, router_logits, w13_weight, w2_weight):
    tokens, hidden = hidden_states.shape
    num_experts = w13_weight.shape[0]
    inter = w2_weight.shape[2]
    coeff = _routing_coeff(router_logits)
    return pl.pallas_call(
        _moe_kernel,
        grid=(num_experts,),
        in_specs=[
            pl.BlockSpec((tokens, hidden), lambda e: (0, 0)),
            pl.BlockSpec((1, 1, tokens), lambda e: (e, 0, 0)),
            pl.BlockSpec((1, 2 * inter, hidden), lambda e: (e, 0, 0)),
            pl.BlockSpec((1, hidden, inter), lambda e: (e, 0, 0)),
        ],
        out_specs=pl.BlockSpec((tokens, hidden), lambda e: (0, 0)),
        out_shape=jax.ShapeDtypeStruct((tokens, hidden), jnp.float32),
    )(hidden_states, coeff, w13_weight, w2_weight)


# speedup vs baseline: 1.3539x; 1.3539x over previous
import jax
import jax.numpy as jnp
from jax.experimental import pallas as pl


def _moe_kernel(x_ref, logits_ref, w13_ref, w2_ref, out_ref):
    e = pl.program_id(0)

    logits = logits_ref[...]
    m1 = jnp.max(logits, axis=-1, keepdims=True)
    idx1 = jnp.argmax(logits, axis=-1, keepdims=True)
    neg = jnp.finfo(jnp.float32).min
    cols = jax.lax.broadcasted_iota(jnp.int32, logits.shape, 1)
    masked = jnp.where(cols == idx1, neg, logits)
    m2 = jnp.max(masked, axis=-1, keepdims=True)
    idx2 = jnp.argmax(masked, axis=-1, keepdims=True)
    r = jnp.exp(m2 - m1)
    w1 = 1.0 / (1.0 + r)
    w2 = r / (1.0 + r)
    coeff = jnp.where(idx1 == e, w1, 0.0) + jnp.where(idx2 == e, w2, 0.0)

    @pl.when(e == 0)
    def _init():
        out_ref[...] = jnp.zeros_like(out_ref)

    x = x_ref[...]
    w13 = w13_ref[0]
    w2m = w2_ref[0]
    inter = w2m.shape[1]
    gate_up = jax.lax.dot_general(
        x, w13, (((1,), (1,)), ((), ())), preferred_element_type=jnp.float32
    )
    gate = gate_up[:, :inter]
    up = gate_up[:, inter:]
    h = gate * jax.nn.sigmoid(gate) * up
    y = jax.lax.dot_general(
        h, w2m, (((1,), (1,)), ((), ())), preferred_element_type=jnp.float32
    )
    out_ref[...] += coeff * y


def kernel(hidden_states, router_logits, w13_weight, w2_weight):
    tokens, hidden = hidden_states.shape
    num_experts = w13_weight.shape[0]
    inter = w2_weight.shape[2]
    return pl.pallas_call(
        _moe_kernel,
        grid=(num_experts,),
        in_specs=[
            pl.BlockSpec((tokens, hidden), lambda e: (0, 0)),
            pl.BlockSpec((tokens, num_experts), lambda e: (0, 0)),
            pl.BlockSpec((1, 2 * inter, hidden), lambda e: (e, 0, 0)),
            pl.BlockSpec((1, hidden, inter), lambda e: (e, 0, 0)),
        ],
        out_specs=pl.BlockSpec((tokens, hidden), lambda e: (0, 0)),
        out_shape=jax.ShapeDtypeStruct((tokens, hidden), jnp.float32),
    )(hidden_states, router_logits, w13_weight, w2_weight)
